# Initial kernel scaffold; baseline (speedup 1.0000x reference)
import jax
import jax.numpy as jnp
from jax.experimental import pallas as pl


def kernel(x, table):
    raise NotImplementedError("write your pallas kernel here")


# speedup vs baseline: 1.7970x; 1.7970x over previous
import functools

import jax
import jax.numpy as jnp
from jax import lax
from jax.experimental import pallas as pl
from jax.experimental.pallas import tpu as pltpu
from jax.experimental.pallas import tpu_sc as plsc


def _emb_gather(x_flat, table, n_per_w, chunk, num_cores):
    n = x_flat.shape[0]
    d = table.shape[1]
    n_chunks = n_per_w // chunk

    mesh = plsc.VectorSubcoreMesh(core_axis_name="c", subcore_axis_name="s")

    @functools.partial(
        pl.kernel,
        mesh=mesh,
        out_type=jax.ShapeDtypeStruct((n, d), jnp.float32),
        scratch_types=[
            pltpu.VMEM((chunk,), jnp.int32),
            pltpu.VMEM((chunk, d), jnp.float32),
            pltpu.SemaphoreType.DMA,
        ],
        compiler_params=pltpu.CompilerParams(use_tc_tiling_on_sc=False),
    )
    def emb(idx_hbm, table_hbm, out_hbm, idx_v, rows_v, sem):
        wid = lax.axis_index("s") * num_cores + lax.axis_index("c")
        base = wid * n_per_w

        def body(i, carry):
            off = base + i * chunk
            pltpu.sync_copy(idx_hbm.at[pl.ds(off, chunk)], idx_v)
            pltpu.async_copy(table_hbm.at[idx_v], rows_v, sem).wait()
            pltpu.sync_copy(rows_v, out_hbm.at[pl.ds(off, chunk)])
            return carry

        lax.fori_loop(0, n_chunks, body, 0)

    return emb(x_flat, table)


def kernel(x, table):
    b, l = x.shape
    d = table.shape[1]
    n = b * l

    info = plsc.get_sparse_core_info()
    nw = info.num_cores * info.num_subcores
    n_per_w = n // nw
    chunk = 512

    x_flat = x.reshape(-1).astype(jnp.int32)
    out = _emb_gather(x_flat, table, n_per_w, chunk, info.num_cores)
    return out.reshape(b, l, d)


# speedup vs baseline: 1.8754x; 1.0436x over previous
import functools

import jax
import jax.numpy as jnp
from jax import lax
from jax.experimental import pallas as pl
from jax.experimental.pallas import tpu as pltpu
from jax.experimental.pallas import tpu_sc as plsc

_NBUF = 2


def _emb_gather(x_flat, table, n_per_w, chunk, num_cores):
    n = x_flat.shape[0]
    d = table.shape[1]
    n_chunks = n_per_w // chunk
    n_groups = n_chunks // _NBUF

    mesh = plsc.VectorSubcoreMesh(core_axis_name="c", subcore_axis_name="s")

    @functools.partial(
        pl.kernel,
        mesh=mesh,
        out_type=jax.ShapeDtypeStruct((n, d), jnp.float32),
        scratch_types=[
            pltpu.VMEM((n_per_w,), jnp.int32),
            pltpu.VMEM((_NBUF, chunk, d), jnp.float32),
            pltpu.SemaphoreType.DMA((_NBUF,)),
            pltpu.SemaphoreType.DMA((_NBUF,)),
        ],
        compiler_params=pltpu.CompilerParams(use_tc_tiling_on_sc=False),
    )
    def emb(idx_hbm, table_hbm, out_hbm, idx_v, rows_v, gsem, wsem):
        wid = lax.axis_index("s") * num_cores + lax.axis_index("c")
        base = wid * n_per_w

        pltpu.sync_copy(idx_hbm.at[pl.ds(base, n_per_w)], idx_v)

        def idx_slice(ci):
            return idx_v.at[pl.ds(pl.multiple_of(ci * chunk, 8), chunk)]

        def start_gather(ci, b):
            pltpu.async_copy(table_hbm.at[idx_slice(ci)], rows_v.at[b],
                             gsem.at[b])

        def wait_gather(ci, b):
            pltpu.make_async_copy(table_hbm.at[idx_slice(ci)], rows_v.at[b],
                                  gsem.at[b]).wait()

        def out_slice(ci):
            return out_hbm.at[pl.ds(base + ci * chunk, chunk)]

        def start_wb(ci, b):
            pltpu.async_copy(rows_v.at[b], out_slice(ci), wsem.at[b])

        def wait_wb(ci, b):
            pltpu.make_async_copy(rows_v.at[b], out_slice(ci),
                                  wsem.at[b]).wait()

        for b in range(_NBUF):
            start_gather(b, b)

        def body(g, carry):
            for b in range(_NBUF):
                i = g * _NBUF + b
                wait_gather(i, b)
                start_wb(i, b)
                wait_wb(i, b)
                start_gather(i + _NBUF, b)
            return carry

        lax.fori_loop(0, n_groups - 1, body, 0)

        for b in range(_NBUF):
            i = (n_groups - 1) * _NBUF + b
            wait_gather(i, b)
            start_wb(i, b)
        for b in range(_NBUF):
            i = (n_groups - 1) * _NBUF + b
            wait_wb(i, b)

    return emb(x_flat, table)


def kernel(x, table):
    b, l = x.shape
    d = table.shape[1]
    n = b * l

    info = plsc.get_sparse_core_info()
    nw = info.num_cores * info.num_subcores
    n_per_w = n // nw
    chunk = 640

    x_flat = x.reshape(-1).astype(jnp.int32)
    out = _emb_gather(x_flat, table, n_per_w, chunk, info.num_cores)
    return out.reshape(b, l, d)
